# baseline (speedup 1.0000x reference)
import functools

import jax
import jax.numpy as jnp
from jax import lax
from jax.experimental import pallas as pl
from jax.experimental.pallas import tpu as pltpu
from jax.experimental.pallas import tpu_sc as plsc

_NC = 2
_NS = 16
_NW = _NC * _NS
_L = 16
_COPY_STRIPS = 10


def _rot_perm(lane, r):
  return (lane + r) & (_L - 1)


def _gather_lanes(x, perm):
  return jnp.take_along_axis(x, perm, axis=0,
                             mode=lax.GatherScatterMode.PROMISE_IN_BOUNDS)


def _tc_body_copy(m, mp, b, d):
  rows_per = m // _COPY_STRIPS

  def body(bx_ref, by_ref, bt_ref, ox_ref, oy_ref, ot_ref, sem):
    for s in range(_COPY_STRIPS):
      pltpu.make_async_copy(
          bx_ref.at[pl.ds(s * rows_per, rows_per)],
          ox_ref.at[pl.ds(s * rows_per, rows_per)],
          sem,
      ).start()
    pltpu.make_async_copy(by_ref, oy_ref.at[pl.ds(0, mp)], sem).start()
    pltpu.make_async_copy(bt_ref, ot_ref.at[pl.ds(0, mp)], sem).start()
    for s in range(_COPY_STRIPS):
      pltpu.make_async_copy(
          bx_ref.at[pl.ds(s * rows_per, rows_per)],
          ox_ref.at[pl.ds(s * rows_per, rows_per)],
          sem,
      ).wait()
    pltpu.make_async_copy(by_ref, oy_ref.at[pl.ds(0, mp)], sem).wait()
    pltpu.make_async_copy(bt_ref, ot_ref.at[pl.ds(0, mp)], sem).wait()

  return pl.pallas_call(
      body,
      in_specs=[pl.BlockSpec(memory_space=pl.ANY)] * 3,
      out_specs=[pl.BlockSpec(memory_space=pl.ANY)] * 3,
      out_shape=(jax.ShapeDtypeStruct((m + b, d), jnp.float32),
                 jax.ShapeDtypeStruct((m + b,), jnp.int32),
                 jax.ShapeDtypeStruct((m + b,), jnp.int32)),
      scratch_shapes=[pltpu.SemaphoreType.DMA],
  )


def _sc_sparse(m, b, d, ox_ref, oy_ref, ot_ref):
  nchunk = b // _L
  ipt = b // _NW
  cpt = ipt // _L
  mesh = plsc.VectorSubcoreMesh(
      core_axis_name="c", subcore_axis_name="s", num_cores=_NC,
      num_subcores=_NS)

  @functools.partial(
      pl.kernel,
      out_type=(),
      mesh=mesh,
      scratch_types=[
          pltpu.VMEM((b,), jnp.int32),
          pltpu.VMEM((cpt, _L), jnp.int32),
          pltpu.VMEM((ipt,), jnp.int32),
          pltpu.VMEM((ipt,), jnp.int32),
          pltpu.VMEM((_L, d), jnp.float32),
          pltpu.VMEM((ipt,), jnp.int32),
          pltpu.VMEM((ipt,), jnp.int32),
          pltpu.SemaphoreType.DMA,
      ],
  )
  def sc(bx_hbm, inx_hbm, by_hbm, bt_hbm, iny_hbm, idx_hbm, idx3_hbm,
         itv_hbm,
         idx_v, myidx_v, myflat_v, w_v, rows_v, val_v, tail_v, sem):
    cid = lax.axis_index("c")
    sid = lax.axis_index("s")
    tid = cid * _NS + sid
    base = tid * ipt
    lane = lax.iota(jnp.int32, _L)

    pltpu.sync_copy(idx_hbm, idx_v)
    pltpu.sync_copy(idx_hbm.at[pl.ds(base, ipt)], myflat_v)
    pltpu.sync_copy(idx3_hbm.at[tid], myidx_v)

    for j in range(cpt):
      g = tid * cpt + j
      v = idx_v[pl.ds(g * _L, _L)]
      best = g * _L + lane

      def wbody(c, best, v=v):
        u = idx_v[pl.ds(c * _L, _L)]
        for r in range(_L):
          perm = _rot_perm(lane, r)
          ur = u if r == 0 else _gather_lanes(u, perm)
          jr = c * _L + perm
          upd = jnp.logical_and(ur == v, jr > best)
          best = jnp.where(upd, jr, best)
        return best

      w_v[pl.ds(j * _L, _L)] = lax.fori_loop(0, nchunk, wbody, best)

    for cc in range(cpt):
      pltpu.async_copy(inx_hbm.at[w_v.at[pl.ds(cc * _L, _L)]],
                       rows_v, sem).wait()
      pltpu.async_copy(rows_v, ox_ref.at[myidx_v.at[cc]], sem).wait()
      pltpu.async_copy(bx_hbm.at[myidx_v.at[cc]], rows_v, sem).wait()
      pltpu.sync_copy(
          rows_v,
          ox_ref.at[pl.ds(pl.multiple_of(m + base + cc * _L, 8), _L)])

    tail_at = pl.ds(pl.multiple_of(m + base, 8), ipt)
    pltpu.async_copy(by_hbm.at[myflat_v], tail_v, sem).wait()
    pltpu.sync_copy(tail_v, oy_ref.at[tail_at])
    pltpu.async_copy(bt_hbm.at[myflat_v], tail_v, sem).wait()
    pltpu.sync_copy(tail_v, ot_ref.at[tail_at])
    pltpu.async_copy(iny_hbm.at[w_v], val_v, sem).wait()
    pltpu.async_copy(val_v, oy_ref.at[myflat_v], sem).wait()
    pltpu.sync_copy(itv_hbm, val_v)
    pltpu.async_copy(val_v, ot_ref.at[myflat_v], sem).wait()

  return sc


def kernel(bx, by, bt, in_x, in_y, in_t, swap_idx):
  m = bx.shape[0]
  b = in_x.shape[0]
  d = 1
  for s in bx.shape[1:]:
    d *= s
  mp = ((m + 127) // 128) * 128
  assert b % (_NW * _L) == 0 and m % _COPY_STRIPS == 0 and mp <= m + b

  bx_f = bx.reshape(m, d)
  inx_f = in_x.reshape(b, d)
  idx3 = swap_idx.reshape(_NW, b // _NW // _L, _L)
  itv = jnp.full((b // _NW,), in_t, dtype=jnp.int32)
  pad = jnp.zeros((mp - m,), dtype=jnp.int32)
  by_p = jnp.concatenate([by, pad])
  bt_p = jnp.concatenate([bt, pad])

  body_x, body_y, body_t = _tc_body_copy(m, mp, b, d)(bx_f, by_p, bt_p)
  ox_ref = jax.new_ref(body_x)
  oy_ref = jax.new_ref(body_y)
  ot_ref = jax.new_ref(body_t)
  _sc_sparse(m, b, d, ox_ref, oy_ref, ot_ref)(
      bx_f, inx_f, by, bt, in_y, swap_idx, idx3, itv)
  out_bx = ox_ref[...].reshape((m + b,) + bx.shape[1:])
  return (out_bx, oy_ref[...], ot_ref[...])


# speedup vs baseline: 12.0051x; 12.0051x over previous
import functools

import jax
import jax.numpy as jnp
from jax import lax
from jax.experimental import pallas as pl
from jax.experimental.pallas import tpu as pltpu
from jax.experimental.pallas import tpu_sc as plsc

_NC = 2
_NS = 16
_NW = _NC * _NS
_L = 16
_COPY_STRIPS = 10


def _rot_perm(lane, r):
  return (lane + r) & (_L - 1)


def _gather_lanes(x, perm):
  return jnp.take_along_axis(x, perm, axis=0,
                             mode=lax.GatherScatterMode.PROMISE_IN_BOUNDS)


_BLK = 400


def _tc_body_copy(m, mp, b, d):
  def body(bx_ref, by_ref, bt_ref, ox_ref, oy_ref, ot_ref, sem):
    i = pl.program_id(0)
    ox_ref[...] = bx_ref[...]

    @pl.when(i == 0)
    def _ints():
      pltpu.make_async_copy(by_ref, oy_ref.at[pl.ds(0, mp)], sem).start()
      pltpu.make_async_copy(bt_ref, ot_ref.at[pl.ds(0, mp)], sem).start()
      pltpu.make_async_copy(by_ref, oy_ref.at[pl.ds(0, mp)], sem).wait()
      pltpu.make_async_copy(bt_ref, ot_ref.at[pl.ds(0, mp)], sem).wait()

  return pl.pallas_call(
      body,
      grid=(m // _BLK,),
      in_specs=[pl.BlockSpec((_BLK, d), lambda i: (i, 0)),
                pl.BlockSpec(memory_space=pl.ANY),
                pl.BlockSpec(memory_space=pl.ANY)],
      out_specs=[pl.BlockSpec((_BLK, d), lambda i: (i, 0)),
                 pl.BlockSpec(memory_space=pl.ANY),
                 pl.BlockSpec(memory_space=pl.ANY)],
      out_shape=(jax.ShapeDtypeStruct((m + b, d), jnp.float32),
                 jax.ShapeDtypeStruct((m + b,), jnp.int32),
                 jax.ShapeDtypeStruct((m + b,), jnp.int32)),
      scratch_shapes=[pltpu.SemaphoreType.DMA],
  )


def _sc_sparse(m, b, d, ox_ref, oy_ref, ot_ref):
  nchunk = b // _L
  ipt = b // _NW
  cpt = ipt // _L
  mesh = plsc.VectorSubcoreMesh(
      core_axis_name="c", subcore_axis_name="s", num_cores=_NC,
      num_subcores=_NS)

  @functools.partial(
      pl.kernel,
      out_type=(),
      mesh=mesh,
      scratch_types=[
          pltpu.VMEM((b,), jnp.int32),
          pltpu.VMEM((cpt, _L), jnp.int32),
          pltpu.VMEM((ipt,), jnp.int32),
          pltpu.VMEM((ipt,), jnp.int32),
          pltpu.VMEM((_L, d), jnp.float32),
          pltpu.VMEM((ipt,), jnp.int32),
          pltpu.VMEM((ipt,), jnp.int32),
          pltpu.SemaphoreType.DMA,
      ],
  )
  def sc(bx_hbm, inx_hbm, by_hbm, bt_hbm, iny_hbm, idx_hbm, idx3_hbm,
         itv_hbm,
         idx_v, myidx_v, myflat_v, w_v, rows_v, val_v, tail_v, sem):
    cid = lax.axis_index("c")
    sid = lax.axis_index("s")
    tid = cid * _NS + sid
    base = tid * ipt
    lane = lax.iota(jnp.int32, _L)

    pltpu.sync_copy(idx_hbm, idx_v)
    pltpu.sync_copy(idx_hbm.at[pl.ds(base, ipt)], myflat_v)
    pltpu.sync_copy(idx3_hbm.at[tid], myidx_v)

    for j in range(cpt):
      g = tid * cpt + j
      v = idx_v[pl.ds(g * _L, _L)]
      best = g * _L + lane

      def wbody(c, best, v=v):
        u = idx_v[pl.ds(c * _L, _L)]
        for r in range(_L):
          perm = _rot_perm(lane, r)
          ur = u if r == 0 else _gather_lanes(u, perm)
          jr = c * _L + perm
          upd = jnp.logical_and(ur == v, jr > best)
          best = jnp.where(upd, jr, best)
        return best

      w_v[pl.ds(j * _L, _L)] = lax.fori_loop(0, nchunk, wbody, best)

    for cc in range(cpt):
      pltpu.async_copy(inx_hbm.at[w_v.at[pl.ds(cc * _L, _L)]],
                       rows_v, sem).wait()
      pltpu.async_copy(rows_v, ox_ref.at[myidx_v.at[cc]], sem).wait()
      pltpu.async_copy(bx_hbm.at[myidx_v.at[cc]], rows_v, sem).wait()
      pltpu.sync_copy(
          rows_v,
          ox_ref.at[pl.ds(pl.multiple_of(m + base + cc * _L, 8), _L)])

    tail_at = pl.ds(pl.multiple_of(m + base, 8), ipt)
    pltpu.async_copy(by_hbm.at[myflat_v], tail_v, sem).wait()
    pltpu.sync_copy(tail_v, oy_ref.at[tail_at])
    pltpu.async_copy(bt_hbm.at[myflat_v], tail_v, sem).wait()
    pltpu.sync_copy(tail_v, ot_ref.at[tail_at])
    pltpu.async_copy(iny_hbm.at[w_v], val_v, sem).wait()
    pltpu.async_copy(val_v, oy_ref.at[myflat_v], sem).wait()
    pltpu.sync_copy(itv_hbm, val_v)
    pltpu.async_copy(val_v, ot_ref.at[myflat_v], sem).wait()

  return sc


def kernel(bx, by, bt, in_x, in_y, in_t, swap_idx):
  m = bx.shape[0]
  b = in_x.shape[0]
  d = 1
  for s in bx.shape[1:]:
    d *= s
  mp = ((m + 127) // 128) * 128
  assert b % (_NW * _L) == 0 and m % _COPY_STRIPS == 0 and mp <= m + b

  bx_f = bx.reshape(m, d)
  inx_f = in_x.reshape(b, d)
  idx3 = swap_idx.reshape(_NW, b // _NW // _L, _L)
  itv = jnp.full((b // _NW,), in_t, dtype=jnp.int32)
  pad = jnp.zeros((mp - m,), dtype=jnp.int32)
  by_p = jnp.concatenate([by, pad])
  bt_p = jnp.concatenate([bt, pad])

  body_x, body_y, body_t = _tc_body_copy(m, mp, b, d)(bx_f, by_p, bt_p)
  ox_ref = jax.new_ref(body_x)
  oy_ref = jax.new_ref(body_y)
  ot_ref = jax.new_ref(body_t)
  _sc_sparse(m, b, d, ox_ref, oy_ref, ot_ref)(
      bx_f, inx_f, by, bt, in_y, swap_idx, idx3, itv)
  out_bx = ox_ref[...].reshape((m + b,) + bx.shape[1:])
  return (out_bx, oy_ref[...], ot_ref[...])


# speedup vs baseline: 12.0670x; 1.0052x over previous
import functools

import jax
import jax.numpy as jnp
from jax import lax
from jax.experimental import pallas as pl
from jax.experimental.pallas import tpu as pltpu
from jax.experimental.pallas import tpu_sc as plsc

_NC = 2
_NS = 16
_NW = _NC * _NS
_L = 16
_COPY_STRIPS = 10


def _rot_perm(lane, r):
  return (lane + r) & (_L - 1)


def _gather_lanes(x, perm):
  return jnp.take_along_axis(x, perm, axis=0,
                             mode=lax.GatherScatterMode.PROMISE_IN_BOUNDS)


_BLK = 1000


def _tc_body_copy(m, mp, b, d):
  def body(bx_ref, by_ref, bt_ref, ox_ref, oy_ref, ot_ref, sem):
    i = pl.program_id(0)
    ox_ref[...] = bx_ref[...]

    @pl.when(i == 0)
    def _ints():
      pltpu.make_async_copy(by_ref, oy_ref.at[pl.ds(0, mp)], sem).start()
      pltpu.make_async_copy(bt_ref, ot_ref.at[pl.ds(0, mp)], sem).start()
      pltpu.make_async_copy(by_ref, oy_ref.at[pl.ds(0, mp)], sem).wait()
      pltpu.make_async_copy(bt_ref, ot_ref.at[pl.ds(0, mp)], sem).wait()

  return pl.pallas_call(
      body,
      grid=(m // _BLK,),
      in_specs=[pl.BlockSpec((_BLK, d), lambda i: (i, 0)),
                pl.BlockSpec(memory_space=pl.ANY),
                pl.BlockSpec(memory_space=pl.ANY)],
      out_specs=[pl.BlockSpec((_BLK, d), lambda i: (i, 0)),
                 pl.BlockSpec(memory_space=pl.ANY),
                 pl.BlockSpec(memory_space=pl.ANY)],
      out_shape=(jax.ShapeDtypeStruct((m + b, d), jnp.float32),
                 jax.ShapeDtypeStruct((m + b,), jnp.int32),
                 jax.ShapeDtypeStruct((m + b,), jnp.int32)),
      scratch_shapes=[pltpu.SemaphoreType.DMA],
  )


def _sc_sparse(m, b, d, ox_ref, oy_ref, ot_ref):
  nchunk = b // _L
  ipt = b // _NW
  cpt = ipt // _L
  mesh = plsc.VectorSubcoreMesh(
      core_axis_name="c", subcore_axis_name="s", num_cores=_NC,
      num_subcores=_NS)

  @functools.partial(
      pl.kernel,
      out_type=(),
      mesh=mesh,
      scratch_types=[
          pltpu.VMEM((b,), jnp.int32),
          pltpu.VMEM((cpt, _L), jnp.int32),
          pltpu.VMEM((ipt,), jnp.int32),
          pltpu.VMEM((ipt,), jnp.int32),
          pltpu.VMEM((_L, d), jnp.float32),
          pltpu.VMEM((ipt,), jnp.int32),
          pltpu.VMEM((ipt,), jnp.int32),
          pltpu.SemaphoreType.DMA,
      ],
  )
  def sc(bx_hbm, inx_hbm, by_hbm, bt_hbm, iny_hbm, idx_hbm, idx3_hbm,
         itv_hbm,
         idx_v, myidx_v, myflat_v, w_v, rows_v, val_v, tail_v, sem):
    cid = lax.axis_index("c")
    sid = lax.axis_index("s")
    tid = cid * _NS + sid
    base = tid * ipt
    lane = lax.iota(jnp.int32, _L)

    pltpu.sync_copy(idx_hbm, idx_v)
    pltpu.sync_copy(idx_hbm.at[pl.ds(base, ipt)], myflat_v)
    pltpu.sync_copy(idx3_hbm.at[tid], myidx_v)

    for j in range(cpt):
      g = tid * cpt + j
      v = idx_v[pl.ds(g * _L, _L)]
      best = g * _L + lane

      def wbody(c, best, v=v):
        u = idx_v[pl.ds(c * _L, _L)]
        for r in range(_L):
          perm = _rot_perm(lane, r)
          ur = u if r == 0 else _gather_lanes(u, perm)
          jr = c * _L + perm
          upd = jnp.logical_and(ur == v, jr > best)
          best = jnp.where(upd, jr, best)
        return best

      w_v[pl.ds(j * _L, _L)] = lax.fori_loop(0, nchunk, wbody, best)

    for cc in range(cpt):
      pltpu.async_copy(inx_hbm.at[w_v.at[pl.ds(cc * _L, _L)]],
                       rows_v, sem).wait()
      pltpu.async_copy(rows_v, ox_ref.at[myidx_v.at[cc]], sem).wait()
      pltpu.async_copy(bx_hbm.at[myidx_v.at[cc]], rows_v, sem).wait()
      pltpu.sync_copy(
          rows_v,
          ox_ref.at[pl.ds(pl.multiple_of(m + base + cc * _L, 8), _L)])

    tail_at = pl.ds(pl.multiple_of(m + base, 8), ipt)
    pltpu.async_copy(by_hbm.at[myflat_v], tail_v, sem).wait()
    pltpu.sync_copy(tail_v, oy_ref.at[tail_at])
    pltpu.async_copy(bt_hbm.at[myflat_v], tail_v, sem).wait()
    pltpu.sync_copy(tail_v, ot_ref.at[tail_at])
    pltpu.async_copy(iny_hbm.at[w_v], val_v, sem).wait()
    pltpu.async_copy(val_v, oy_ref.at[myflat_v], sem).wait()
    pltpu.sync_copy(itv_hbm, val_v)
    pltpu.async_copy(val_v, ot_ref.at[myflat_v], sem).wait()

  return sc


def kernel(bx, by, bt, in_x, in_y, in_t, swap_idx):
  m = bx.shape[0]
  b = in_x.shape[0]
  d = 1
  for s in bx.shape[1:]:
    d *= s
  mp = ((m + 127) // 128) * 128
  assert b % (_NW * _L) == 0 and m % _COPY_STRIPS == 0 and mp <= m + b

  bx_f = bx.reshape(m, d)
  inx_f = in_x.reshape(b, d)
  idx3 = swap_idx.reshape(_NW, b // _NW // _L, _L)
  itv = jnp.full((b // _NW,), in_t, dtype=jnp.int32)
  pad = jnp.zeros((mp - m,), dtype=jnp.int32)
  by_p = jnp.concatenate([by, pad])
  bt_p = jnp.concatenate([bt, pad])

  body_x, body_y, body_t = _tc_body_copy(m, mp, b, d)(bx_f, by_p, bt_p)
  ox_ref = jax.new_ref(body_x)
  oy_ref = jax.new_ref(body_y)
  ot_ref = jax.new_ref(body_t)
  _sc_sparse(m, b, d, ox_ref, oy_ref, ot_ref)(
      bx_f, inx_f, by, bt, in_y, swap_idx, idx3, itv)
  out_bx = ox_ref[...].reshape((m + b,) + bx.shape[1:])
  return (out_bx, oy_ref[...], ot_ref[...])
